# baseline (speedup 1.0000x reference)
import functools

import jax
import jax.numpy as jnp
from jax import lax
from jax.experimental import pallas as pl
from jax.experimental.pallas import tpu as pltpu
from jax.experimental.pallas import tpu_sc as plsc

_NC = 2
_NS = 16
_NW = _NC * _NS
_LANES = 16
_CH = 100


def _vrecip(x):
    r = plsc.bitcast(0x7EF311C3 - plsc.bitcast(x, jnp.int32), jnp.float32)
    r = r * (2.0 - x * r)
    r = r * (2.0 - x * r)
    r = r * (2.0 - x * r)
    return r


def _make_sc_pool(B, Lw, Ln, D):
    T = Lw + Ln
    EPW = B // _NW
    NPAIR = EPW // 2
    NG = T // _LANES
    KD = D // _LANES
    ncw = Lw // _CH
    ncn = Ln // _CH

    mesh = plsc.VectorSubcoreMesh(
        core_axis_name="c", subcore_axis_name="s",
        num_cores=_NC, num_subcores=_NS)

    @functools.partial(
        pl.kernel,
        mesh=mesh,
        compiler_params=pltpu.CompilerParams(
            needs_layout_passes=False, use_tc_tiling_on_sc=False),
        out_type=jax.ShapeDtypeStruct((B, 80), jnp.float32),
        scratch_types=[
            pltpu.VMEM((ncw + ncn, _CH), jnp.int32),
            pltpu.VMEM((ncw + ncn, _CH), jnp.int32),
            pltpu.VMEM((T, D), jnp.float32),
            pltpu.VMEM((T, D), jnp.float32),
            pltpu.VMEM((80,), jnp.float32),
            pltpu.VMEM((80,), jnp.float32),
            pltpu.SemaphoreType.DMA,
            pltpu.SemaphoreType.DMA,
            pltpu.SemaphoreType.DMA,
            pltpu.SemaphoreType.DMA,
        ],
    )
    def sc_pool(x0r, x1r, embw, embn, sg, idxv_a, idxv_b, rows_a, rows_b,
                st_a, st_b, gs_a, gs_b, ss_a, ss_b):
        wid = lax.axis_index("s") * _NC + lax.axis_index("c")
        base = wid * EPW

        def fire(elem, idxv, rowsv, gsem):
            pltpu.sync_copy(x0r.at[elem], idxv.at[pl.ds(0, ncw)])
            pltpu.sync_copy(x1r.at[elem], idxv.at[pl.ds(ncw, ncn)])
            for j in range(ncw):
                pltpu.async_copy(embw.at[idxv.at[j]],
                                 rowsv.at[pl.ds(j * _CH, _CH)], gsem)
            for j in range(ncn):
                pltpu.async_copy(embn.at[idxv.at[ncw + j]],
                                 rowsv.at[pl.ds(Lw + j * _CH, _CH)], gsem)

        def wait_gather(idxv, rowsv, gsem):
            for j in range(ncw):
                pltpu.make_async_copy(embw.at[idxv.at[j]],
                                      rowsv.at[pl.ds(j * _CH, _CH)],
                                      gsem).wait()
            for j in range(ncn):
                pltpu.make_async_copy(embn.at[idxv.at[ncw + j]],
                                      rowsv.at[pl.ds(Lw + j * _CH, _CH)],
                                      gsem).wait()

        iota16 = lax.iota(jnp.int32, _LANES)

        def compute(elem, rowsv, stv, ssem):
            zero = jnp.zeros((_LANES,), jnp.float32)

            def group(g, carry):
                acc = list(carry[:KD])
                gacc = carry[KD]
                n16 = zero
                for tt in range(_LANES):
                    t = g * _LANES + tt
                    v = [rowsv[t, pl.ds(k * _LANES, _LANES)]
                         for k in range(KD)]
                    sq = v[0] * v[0]
                    for k in range(1, KD):
                        sq = sq + v[k] * v[k]
                    n16 = jnp.where(iota16 == tt, jnp.sum(sq), n16)
                den = 1.0 + n16
                dist = jnp.abs(1.0 - n16)
                gamma = jnp.minimum(den * _vrecip(dist), 1000.0)
                gacc = gacc + gamma
                w16 = 2.0 * gamma * _vrecip(den)
                for tt in range(_LANES):
                    t = g * _LANES + tt
                    wv = jnp.full((_LANES,), w16[tt])
                    for k in range(KD):
                        acc[k] = acc[k] + wv * rowsv[t, pl.ds(k * _LANES,
                                                              _LANES)]
                return (*acc, gacc)

            carry = lax.fori_loop(0, NG, group, (zero,) * (KD + 1))
            for k in range(KD):
                stv[pl.ds(k * _LANES, _LANES)] = carry[k]
            stv[pl.ds(D, _LANES)] = jnp.full((_LANES,), jnp.sum(carry[KD]))
            pltpu.async_copy(stv, sg.at[elem], ssem)

        def wait_store(stv, ssem):
            pltpu.make_async_copy(stv, sg.at[base], ssem).wait()

        fire(base, idxv_a, rows_a, gs_a)

        def pair(p, carry):
            i0 = base + 2 * p
            fire(i0 + 1, idxv_b, rows_b, gs_b)
            wait_gather(idxv_a, rows_a, gs_a)

            @pl.when(p > 0)
            def _():
                wait_store(st_a, ss_a)

            compute(i0, rows_a, st_a, ss_a)

            @pl.when(p < NPAIR - 1)
            def _():
                fire(i0 + 2, idxv_a, rows_a, gs_a)

            wait_gather(idxv_b, rows_b, gs_b)

            @pl.when(p > 0)
            def _():
                wait_store(st_b, ss_b)

            compute(i0 + 1, rows_b, st_b, ss_b)
            return carry

        lax.fori_loop(0, NPAIR, pair, 0)
        wait_store(st_a, ss_a)
        wait_store(st_b, ss_b)

    return sc_pool


def _artanh(x):
    x = jnp.clip(x, -1.0 + 1e-7, 1.0 - 1e-7)
    return 0.5 * jnp.log((1.0 + x) / (1.0 - x))


def _make_tc_head(B, D, C):
    BLK = 512

    def body(sg_ref, w_ref, b_ref, o_ref):
        sgb = sg_ref[...]
        s = sgb[:, :D]
        g = sgb[:, D:D + 1]
        mk = s / jnp.clip(g, 1e-15, None)
        m2 = jnp.clip(jnp.sum(mk * mk, -1, keepdims=True), 0.0, 1.0 - 1e-6)
        mid = mk / (1.0 + jnp.sqrt(1.0 - m2))
        xn = jnp.sqrt(jnp.clip(jnp.sum(mid * mid, -1, keepdims=True),
                               1e-15, None))
        mx = lax.dot_general(mid, w_ref[...], (((1,), (1,)), ((), ())),
                             preferred_element_type=jnp.float32)
        mxn = jnp.sqrt(jnp.clip(jnp.sum(mx * mx, -1, keepdims=True),
                                1e-15, None))
        res = jnp.tanh(mxn / xn * _artanh(xn)) * mx / mxn
        res = jnp.where(jnp.sum(jnp.abs(mx), -1, keepdims=True) < 1e-10,
                        jnp.zeros_like(res), res)
        bv = b_ref[...]
        bn = jnp.sqrt(jnp.clip(jnp.sum(bv * bv, -1, keepdims=True),
                               1e-15, None))
        bb = jnp.tanh(bn) * bv / bn
        x2 = jnp.sum(res * res, -1, keepdims=True)
        y2 = jnp.sum(bb * bb, -1, keepdims=True)
        xy = jnp.sum(res * bb, -1, keepdims=True)
        num = (1.0 + 2.0 * xy + y2) * res + (1.0 - x2) * bb
        h = num / jnp.clip(1.0 + 2.0 * xy + x2 * y2, 1e-15, None)
        hn = jnp.sqrt(jnp.clip(jnp.sum(h * h, -1, keepdims=True),
                               1e-15, None))
        h = jnp.where(hn > 1.0 - 1e-5, h / hn * (1.0 - 1e-5), h)
        hn2 = jnp.sqrt(jnp.clip(jnp.sum(h * h, -1, keepdims=True),
                                1e-15, None))
        o_ref[...] = _artanh(hn2) * h / hn2

    return pl.pallas_call(
        body,
        out_shape=jax.ShapeDtypeStruct((B, C), jnp.float32),
        grid=(B // BLK,),
        in_specs=[
            pl.BlockSpec((BLK, 80), lambda i: (i, 0)),
            pl.BlockSpec((C, D), lambda i: (0, 0)),
            pl.BlockSpec((1, C), lambda i: (0, 0)),
        ],
        out_specs=pl.BlockSpec((BLK, C), lambda i: (i, 0)),
    )


def kernel(x0, x1, emb_word, emb_ngram, W, b):
    B, Lw = x0.shape
    Ln = x1.shape[1]
    D = emb_word.shape[1]
    C = W.shape[0]
    x0r = x0.astype(jnp.int32).reshape(B, Lw // _CH, _CH)
    x1r = x1.astype(jnp.int32).reshape(B, Ln // _CH, _CH)
    sg = _make_sc_pool(B, Lw, Ln, D)(x0r, x1r, emb_word, emb_ngram)
    return _make_tc_head(B, D, C)(sg, W, b.reshape(1, C))


# speedup vs baseline: 1.1844x; 1.1844x over previous
import functools

import jax
import jax.numpy as jnp
from jax import lax
from jax.experimental import pallas as pl
from jax.experimental.pallas import tpu as pltpu
from jax.experimental.pallas import tpu_sc as plsc

_NC = 2
_NS = 16
_NW = _NC * _NS
_LANES = 16
_CH = 100


def _srecip(x):
    r = lax.bitcast_convert_type(0x7EF311C3 - lax.bitcast_convert_type(
        x, jnp.int32), jnp.float32)
    r = r * (2.0 - x * r)
    r = r * (2.0 - x * r)
    r = r * (2.0 - x * r)
    return r


def _make_sc_pool(B, Lw, Ln, D):
    T = Lw + Ln
    EPW = B // _NW
    NPAIR = EPW // 2
    NG = T // _LANES
    KD = D // _LANES
    ncw = Lw // _CH
    ncn = Ln // _CH

    mesh = plsc.VectorSubcoreMesh(
        core_axis_name="c", subcore_axis_name="s",
        num_cores=_NC, num_subcores=_NS)

    @functools.partial(
        pl.kernel,
        mesh=mesh,
        compiler_params=pltpu.CompilerParams(
            needs_layout_passes=False, use_tc_tiling_on_sc=False),
        out_type=jax.ShapeDtypeStruct((B, 80), jnp.float32),
        scratch_types=[
            pltpu.VMEM((ncw + ncn, _CH), jnp.int32),
            pltpu.VMEM((ncw + ncn, _CH), jnp.int32),
            pltpu.VMEM((T, D), jnp.float32),
            pltpu.VMEM((T, D), jnp.float32),
            pltpu.VMEM((80,), jnp.float32),
            pltpu.VMEM((80,), jnp.float32),
            pltpu.SemaphoreType.DMA,
            pltpu.SemaphoreType.DMA,
            pltpu.SemaphoreType.DMA,
            pltpu.SemaphoreType.DMA,
        ],
    )
    def sc_pool(x0r, x1r, embw, embn, sg, idxv_a, idxv_b, rows_a, rows_b,
                st_a, st_b, gs_a, gs_b, ss_a, ss_b):
        wid = lax.axis_index("s") * _NC + lax.axis_index("c")
        base = wid * EPW

        def fire(elem, idxv, rowsv, gsem):
            pltpu.sync_copy(x0r.at[elem], idxv.at[pl.ds(0, ncw)])
            pltpu.sync_copy(x1r.at[elem], idxv.at[pl.ds(ncw, ncn)])
            for j in range(ncw):
                pltpu.async_copy(embw.at[idxv.at[j]],
                                 rowsv.at[pl.ds(j * _CH, _CH)], gsem)
            for j in range(ncn):
                pltpu.async_copy(embn.at[idxv.at[ncw + j]],
                                 rowsv.at[pl.ds(Lw + j * _CH, _CH)], gsem)

        def wait_gather(idxv, rowsv, gsem):
            for j in range(ncw):
                pltpu.make_async_copy(embw.at[idxv.at[j]],
                                      rowsv.at[pl.ds(j * _CH, _CH)],
                                      gsem).wait()
            for j in range(ncn):
                pltpu.make_async_copy(embn.at[idxv.at[ncw + j]],
                                      rowsv.at[pl.ds(Lw + j * _CH, _CH)],
                                      gsem).wait()

        iota16 = lax.iota(jnp.int32, _LANES)

        def compute(elem, rowsv, stv, ssem):
            zero = jnp.zeros((_LANES,), jnp.float32)

            def group(g, carry):
                acc = list(carry[:KD])
                gtot = carry[KD]
                for tt in range(_LANES):
                    t = g * _LANES + tt
                    v = [rowsv[t, pl.ds(k * _LANES, _LANES)]
                         for k in range(KD)]
                    sq = v[0] * v[0]
                    for k in range(1, KD):
                        sq = sq + v[k] * v[k]
                    n = jnp.sum(sq)
                    den = 1.0 + n
                    dist = jnp.maximum(jnp.abs(1.0 - n), 1e-30)
                    r2 = _srecip(den * dist)
                    gamma = jnp.minimum(den * den * r2, 1000.0)
                    gtot = gtot + gamma
                    wv = jnp.full((_LANES,), (2.0 * gamma) * (dist * r2))
                    for k in range(KD):
                        acc[k] = acc[k] + wv * v[k]
                return (*acc, gtot)

            carry = lax.fori_loop(0, NG, group,
                                  (zero,) * KD + (jnp.float32(0.0),))
            for k in range(KD):
                stv[pl.ds(k * _LANES, _LANES)] = carry[k]
            stv[pl.ds(D, _LANES)] = jnp.full((_LANES,), carry[KD])
            pltpu.async_copy(stv, sg.at[elem], ssem)

        def wait_store(stv, ssem):
            pltpu.make_async_copy(stv, sg.at[base], ssem).wait()

        fire(base, idxv_a, rows_a, gs_a)

        def pair(p, carry):
            i0 = base + 2 * p
            fire(i0 + 1, idxv_b, rows_b, gs_b)
            wait_gather(idxv_a, rows_a, gs_a)

            @pl.when(p > 0)
            def _():
                wait_store(st_a, ss_a)

            compute(i0, rows_a, st_a, ss_a)

            @pl.when(p < NPAIR - 1)
            def _():
                fire(i0 + 2, idxv_a, rows_a, gs_a)

            wait_gather(idxv_b, rows_b, gs_b)

            @pl.when(p > 0)
            def _():
                wait_store(st_b, ss_b)

            compute(i0 + 1, rows_b, st_b, ss_b)
            return carry

        lax.fori_loop(0, NPAIR, pair, 0)
        wait_store(st_a, ss_a)
        wait_store(st_b, ss_b)

    return sc_pool


def _artanh(x):
    x = jnp.clip(x, -1.0 + 1e-7, 1.0 - 1e-7)
    return 0.5 * jnp.log((1.0 + x) / (1.0 - x))


def _make_tc_head(B, D, C):
    BLK = 512

    def body(sg_ref, w_ref, b_ref, o_ref):
        sgb = sg_ref[...]
        s = sgb[:, :D]
        g = sgb[:, D:D + 1]
        mk = s / jnp.clip(g, 1e-15, None)
        m2 = jnp.clip(jnp.sum(mk * mk, -1, keepdims=True), 0.0, 1.0 - 1e-6)
        mid = mk / (1.0 + jnp.sqrt(1.0 - m2))
        xn = jnp.sqrt(jnp.clip(jnp.sum(mid * mid, -1, keepdims=True),
                               1e-15, None))
        mx = lax.dot_general(mid, w_ref[...], (((1,), (1,)), ((), ())),
                             preferred_element_type=jnp.float32)
        mxn = jnp.sqrt(jnp.clip(jnp.sum(mx * mx, -1, keepdims=True),
                                1e-15, None))
        res = jnp.tanh(mxn / xn * _artanh(xn)) * mx / mxn
        res = jnp.where(jnp.sum(jnp.abs(mx), -1, keepdims=True) < 1e-10,
                        jnp.zeros_like(res), res)
        bv = b_ref[...]
        bn = jnp.sqrt(jnp.clip(jnp.sum(bv * bv, -1, keepdims=True),
                               1e-15, None))
        bb = jnp.tanh(bn) * bv / bn
        x2 = jnp.sum(res * res, -1, keepdims=True)
        y2 = jnp.sum(bb * bb, -1, keepdims=True)
        xy = jnp.sum(res * bb, -1, keepdims=True)
        num = (1.0 + 2.0 * xy + y2) * res + (1.0 - x2) * bb
        h = num / jnp.clip(1.0 + 2.0 * xy + x2 * y2, 1e-15, None)
        hn = jnp.sqrt(jnp.clip(jnp.sum(h * h, -1, keepdims=True),
                               1e-15, None))
        h = jnp.where(hn > 1.0 - 1e-5, h / hn * (1.0 - 1e-5), h)
        hn2 = jnp.sqrt(jnp.clip(jnp.sum(h * h, -1, keepdims=True),
                                1e-15, None))
        o_ref[...] = _artanh(hn2) * h / hn2

    return pl.pallas_call(
        body,
        out_shape=jax.ShapeDtypeStruct((B, C), jnp.float32),
        grid=(B // BLK,),
        in_specs=[
            pl.BlockSpec((BLK, 80), lambda i: (i, 0)),
            pl.BlockSpec((C, D), lambda i: (0, 0)),
            pl.BlockSpec((1, C), lambda i: (0, 0)),
        ],
        out_specs=pl.BlockSpec((BLK, C), lambda i: (i, 0)),
    )


def kernel(x0, x1, emb_word, emb_ngram, W, b):
    B, Lw = x0.shape
    Ln = x1.shape[1]
    D = emb_word.shape[1]
    C = W.shape[0]
    x0r = x0.astype(jnp.int32).reshape(B, Lw // _CH, _CH)
    x1r = x1.astype(jnp.int32).reshape(B, Ln // _CH, _CH)
    sg = _make_sc_pool(B, Lw, Ln, D)(x0r, x1r, emb_word, emb_ngram)
    return _make_tc_head(B, D, C)(sg, W, b.reshape(1, C))
